# Initial kernel scaffold; baseline (speedup 1.0000x reference)
import functools

import jax
import jax.numpy as jnp
from jax import lax
from jax.experimental import pallas as pl
from jax.experimental.pallas import tpu as pltpu
from jax.experimental.pallas import tpu_sc as plsc

N_NODES = 10000
N_EDGES = 160000
D = 256
H = D // 2
CHUNK = 128
N_CHUNKS = N_EDGES // CHUNK
N_TILES = 16
ROWS_PER_TILE = N_NODES // N_TILES
MAX_CHUNKS_PER_TILE = (N_CHUNKS + N_TILES - 1) // N_TILES


def _sc_segment_sum(xh, idx_packed, zeros):
    mesh = plsc.VectorSubcoreMesh(core_axis_name="c", subcore_axis_name="s")

    @functools.partial(
        pl.kernel,
        out_type=jax.ShapeDtypeStruct((2 * N_NODES, H), jnp.float32),
        mesh=mesh,
        scratch_types=[
            pltpu.VMEM((2, CHUNK), jnp.int32),
            pltpu.VMEM((CHUNK, H), jnp.float32),
            pltpu.VMEM_SHARED((N_NODES, H), jnp.float32),
        ],
    )
    def sc_kernel(xh_hbm, idx_hbm, zeros_hbm, out_hbm, idx_v, rows_v, acc_sh):
        c = lax.axis_index("c")
        s = lax.axis_index("s")
        pltpu.sync_copy(zeros_hbm, acc_sh.at[pl.ds(s * ROWS_PER_TILE, ROWS_PER_TILE)])
        plsc.subcore_barrier()

        @pl.loop(0, MAX_CHUNKS_PER_TILE)
        def _(k):
            i = k * N_TILES + s

            @pl.when(i < N_CHUNKS)
            def _():
                pltpu.sync_copy(idx_hbm.at[c, i], idx_v)
                pltpu.sync_copy(xh_hbm.at[idx_v.at[0]], rows_v)
                pltpu.sync_copy(rows_v, acc_sh.at[idx_v.at[1]], add=True)

        plsc.subcore_barrier()
        pltpu.sync_copy(
            acc_sh.at[pl.ds(s * ROWS_PER_TILE, ROWS_PER_TILE)],
            out_hbm.at[pl.ds(c * N_NODES + s * ROWS_PER_TILE, ROWS_PER_TILE)],
        )

    return sc_kernel(xh, idx_packed, zeros)


def _tc_finish_body(a0_ref, a1_ref, x_ref, w0_ref, w1_ref, wr_ref, b_ref, o_ref):
    y = jnp.dot(a0_ref[...], w0_ref[...],
                preferred_element_type=jnp.float32,
                precision=jax.lax.Precision.HIGHEST)
    y = y + jnp.dot(a1_ref[...], w1_ref[...],
                    preferred_element_type=jnp.float32,
                    precision=jax.lax.Precision.HIGHEST)
    y = y + jnp.dot(x_ref[...], wr_ref[...],
                    preferred_element_type=jnp.float32,
                    precision=jax.lax.Precision.HIGHEST)
    y = y + b_ref[...]
    m = jnp.max(y, axis=-1, keepdims=True)
    t = y - m
    lse = jnp.log(jnp.sum(jnp.exp(t), axis=-1, keepdims=True))
    o_ref[...] = t - lse


def _tc_finish(agg0, agg1, x, w0, w1, wr, b2d):
    n = x.shape[0]
    blk = 1000
    return pl.pallas_call(
        _tc_finish_body,
        grid=(n // blk,),
        in_specs=[
            pl.BlockSpec((blk, H), lambda i: (i, 0)),
            pl.BlockSpec((blk, H), lambda i: (i, 0)),
            pl.BlockSpec((blk, D), lambda i: (i, 0)),
            pl.BlockSpec((H, D), lambda i: (0, 0)),
            pl.BlockSpec((H, D), lambda i: (0, 0)),
            pl.BlockSpec((D, D), lambda i: (0, 0)),
            pl.BlockSpec((1, D), lambda i: (0, 0)),
        ],
        out_specs=pl.BlockSpec((blk, D), lambda i: (i, 0)),
        out_shape=jax.ShapeDtypeStruct((n, D), jnp.float32),
    )(agg0, agg1, x, w0, w1, wr, b2d)


def kernel(x, edge_index, W_rel, W_root, b):
    src = edge_index[0]
    dst = edge_index[1]
    xh = jnp.concatenate([x[:, :H], x[:, H:]], axis=0)
    srcs = src.reshape(N_CHUNKS, CHUNK)
    dsts = dst.reshape(N_CHUNKS, CHUNK)
    core0 = jnp.stack([srcs, dsts], axis=1)
    core1 = jnp.stack([srcs + N_NODES, dsts], axis=1)
    idx_packed = jnp.stack([core0, core1])
    zeros = jnp.zeros((ROWS_PER_TILE, H), jnp.float32)

    agg_cat = _sc_segment_sum(xh, idx_packed, zeros)

    out = _tc_finish(
        agg_cat[:N_NODES], agg_cat[N_NODES:], x,
        W_rel[:, :H].T, W_rel[:, H:].T, W_root.T, b.reshape(1, D),
    )
    return out


# speedup vs baseline: 4.1629x; 4.1629x over previous
import functools

import jax
import jax.numpy as jnp
from jax import lax
from jax.experimental import pallas as pl
from jax.experimental.pallas import tpu as pltpu
from jax.experimental.pallas import tpu_sc as plsc

N_NODES = 10000
N_PAD = 10240
N_EDGES = 160000
D = 256
H = D // 2
CHUNK = 128
N_CHUNKS = N_EDGES // CHUNK
N_TILES = 16
ROWS_PER_TILE = N_PAD // N_TILES
MAX_CHUNKS_PER_TILE = (N_CHUNKS + N_TILES - 1) // N_TILES


def _sc_segment_sum(xh, src_idx, dst_idx, zeros):
    mesh = plsc.VectorSubcoreMesh(core_axis_name="c", subcore_axis_name="s")

    @functools.partial(
        pl.kernel,
        out_type=jax.ShapeDtypeStruct((2 * N_PAD, H), jnp.float32),
        mesh=mesh,
        scratch_types=[
            pltpu.VMEM((CHUNK,), jnp.int32),
            pltpu.VMEM((CHUNK,), jnp.int32),
            pltpu.VMEM((CHUNK, H), jnp.float32),
            pltpu.VMEM_SHARED((N_PAD, H), jnp.float32),
        ],
    )
    def sc_kernel(xh_hbm, src_hbm, dst_hbm, zeros_hbm, out_hbm,
                  src_v, dst_v, rows_v, acc_sh):
        c = lax.axis_index("c")
        s = lax.axis_index("s")
        pltpu.sync_copy(zeros_hbm, acc_sh.at[pl.ds(s * ROWS_PER_TILE, ROWS_PER_TILE)])
        plsc.subcore_barrier()

        @pl.loop(0, MAX_CHUNKS_PER_TILE)
        def _(k):
            i = k * N_TILES + s

            @pl.when(i < N_CHUNKS)
            def _():
                pltpu.sync_copy(src_hbm.at[c, i], src_v)
                pltpu.sync_copy(dst_hbm.at[i], dst_v)
                pltpu.sync_copy(xh_hbm.at[src_v], rows_v)
                pltpu.sync_copy(rows_v, acc_sh.at[dst_v], add=True)

        plsc.subcore_barrier()
        pltpu.sync_copy(
            acc_sh.at[pl.ds(s * ROWS_PER_TILE, ROWS_PER_TILE)],
            out_hbm.at[pl.ds(c * N_PAD + s * ROWS_PER_TILE, ROWS_PER_TILE)],
        )

    return sc_kernel(xh, src_idx, dst_idx, zeros)


def _tc_finish_body(a0_ref, a1_ref, x_ref, w0_ref, w1_ref, wr_ref, b_ref, o_ref):
    y = jnp.dot(a0_ref[...], w0_ref[...],
                preferred_element_type=jnp.float32,
                precision=jax.lax.Precision.HIGHEST)
    y = y + jnp.dot(a1_ref[...], w1_ref[...],
                    preferred_element_type=jnp.float32,
                    precision=jax.lax.Precision.HIGHEST)
    y = y + jnp.dot(x_ref[...], wr_ref[...],
                    preferred_element_type=jnp.float32,
                    precision=jax.lax.Precision.HIGHEST)
    y = y + b_ref[...]
    m = jnp.max(y, axis=-1, keepdims=True)
    t = y - m
    lse = jnp.log(jnp.sum(jnp.exp(t), axis=-1, keepdims=True))
    o_ref[...] = t - lse


def _tc_finish(agg0, agg1, x, w0, w1, wr, b2d):
    n = x.shape[0]
    blk = 1000
    return pl.pallas_call(
        _tc_finish_body,
        grid=(n // blk,),
        in_specs=[
            pl.BlockSpec((blk, H), lambda i: (i, 0)),
            pl.BlockSpec((blk, H), lambda i: (i, 0)),
            pl.BlockSpec((blk, D), lambda i: (i, 0)),
            pl.BlockSpec((H, D), lambda i: (0, 0)),
            pl.BlockSpec((H, D), lambda i: (0, 0)),
            pl.BlockSpec((D, D), lambda i: (0, 0)),
            pl.BlockSpec((1, D), lambda i: (0, 0)),
        ],
        out_specs=pl.BlockSpec((blk, D), lambda i: (i, 0)),
        out_shape=jax.ShapeDtypeStruct((n, D), jnp.float32),
    )(agg0, agg1, x, w0, w1, wr, b2d)


def kernel(x, edge_index, W_rel, W_root, b):
    src = edge_index[0]
    dst = edge_index[1]
    xh = jnp.concatenate([x[:, :H], x[:, H:]], axis=0)
    srcs = src.reshape(N_CHUNKS, CHUNK)
    src_idx = jnp.stack([srcs, srcs + N_NODES])
    dst_idx = dst.reshape(N_CHUNKS, CHUNK)
    zeros = jnp.zeros((ROWS_PER_TILE, H), jnp.float32)

    agg_cat = _sc_segment_sum(xh, src_idx, dst_idx, zeros)

    out = _tc_finish(
        agg_cat[:N_NODES], agg_cat[N_PAD:N_PAD + N_NODES], x,
        W_rel[:, :H].T, W_rel[:, H:].T, W_root.T, b.reshape(1, D),
    )
    return out
